# baseline (speedup 1.0000x reference)
import functools

import jax
import jax.numpy as jnp
from jax import lax
from jax.experimental import pallas as pl
from jax.experimental.pallas import tpu as pltpu
from jax.experimental.pallas import tpu_sc as plsc

T, D_MODEL, D_FF, E, TOP_K = 2048, 768, 2048, 8, 2
TILE_R = 128
R_MAX = T * TOP_K + E * TILE_R
NT = R_MAX // TILE_R

NW = 32


@functools.lru_cache(maxsize=None)
def _sc_mesh():
    return plsc.VectorSubcoreMesh(core_axis_name="c", subcore_axis_name="s")


_SC_PARAMS = pltpu.CompilerParams(needs_layout_passes=False)


def _router_kernel(x_ref, wg_ref, steer_ref, p1_ref, p2_ref, w1_ref,
                   w2_ref, te_ref):
    x = x_ref[...]
    logits = jnp.dot(x, wg_ref[...], preferred_element_type=jnp.float32)
    logits = logits + steer_ref[...]
    eiota = lax.broadcasted_iota(jnp.int32, (T, E), 1)

    m1 = jnp.max(logits, axis=-1, keepdims=True)
    i1 = jnp.min(jnp.where(logits == m1, eiota, E), axis=-1, keepdims=True)
    masked = jnp.where(eiota == i1, jnp.float32(-1e30), logits)
    m2 = jnp.max(masked, axis=-1, keepdims=True)
    i2 = jnp.min(jnp.where(masked == m2, eiota, E), axis=-1, keepdims=True)

    w1 = 1.0 / (1.0 + jnp.exp(m2 - m1))
    w1_ref[...] = w1
    w2_ref[...] = 1.0 - w1

    cnt = ((eiota == i1) | (eiota == i2)).astype(jnp.float32)
    r_io = lax.broadcasted_iota(jnp.int32, (T, T), 0)
    c_io = lax.broadcasted_iota(jnp.int32, (T, T), 1)
    ltri = (r_io > c_io).astype(jnp.float32)
    exc = jnp.dot(ltri, cnt, preferred_element_type=jnp.float32)

    gs = jnp.sum(cnt, axis=0, keepdims=True)
    gs_pad = jnp.ceil(gs / TILE_R) * TILE_R
    ei = lax.broadcasted_iota(jnp.int32, (E, E), 0)
    ej = lax.broadcasted_iota(jnp.int32, (E, E), 1)
    upper = (ei < ej).astype(jnp.float32)
    off = jnp.dot(gs_pad, upper, preferred_element_type=jnp.float32)

    pos = off + exc
    p1 = jnp.sum(jnp.where(eiota == i1, pos, 0.0), axis=-1, keepdims=True)
    p2 = jnp.sum(jnp.where(eiota == i2, pos, 0.0), axis=-1, keepdims=True)
    p1_ref[...] = p1.astype(jnp.int32)
    p2_ref[...] = p2.astype(jnp.int32)

    pad_end = off + gs_pad
    t_start = (lax.broadcasted_iota(jnp.int32, (NT, 1), 0)
               * TILE_R).astype(jnp.float32)
    te_ref[...] = jnp.sum((t_start >= pad_end).astype(jnp.int32), axis=-1,
                          keepdims=True)


_router = pl.pallas_call(
    _router_kernel,
    out_shape=[
        jax.ShapeDtypeStruct((T, 1), jnp.int32),
        jax.ShapeDtypeStruct((T, 1), jnp.int32),
        jax.ShapeDtypeStruct((T, 1), jnp.float32),
        jax.ShapeDtypeStruct((T, 1), jnp.float32),
        jax.ShapeDtypeStruct((NT, 1), jnp.int32),
    ],
)


def _scatter_body(p1_hbm, p2_hbm, w1_hbm, w2_hbm, idx_out, ws_out,
                  p1_v, p2_v, w1_v, w2_v, idx_v, ws_v):
    wid = lax.axis_index("s") * 2 + lax.axis_index("c")

    @pl.when(wid == 0)
    def _():
        pltpu.sync_copy(p1_hbm, p1_v)
        pltpu.sync_copy(p2_hbm, p2_v)
        pltpu.sync_copy(w1_hbm, w1_v)
        pltpu.sync_copy(w2_hbm, w2_v)

        zi = jnp.zeros((16,), jnp.int32)
        zf = jnp.zeros((16,), jnp.float32)

        def zero_body(i, c):
            idx_v[pl.ds(i * 16, 16)] = zi
            ws_v[pl.ds(i * 16, 16)] = zf
            return c

        lax.fori_loop(0, R_MAX // 16, zero_body, 0)

        iota16 = lax.iota(jnp.int32, 16)

        def sc_body(i, c):
            tok = iota16 + i * 16
            pos1 = p1_v[pl.ds(i * 16, 16)]
            plsc.store_scatter(idx_v, [pos1], tok)
            plsc.store_scatter(ws_v, [pos1], w1_v[pl.ds(i * 16, 16)])
            pos2 = p2_v[pl.ds(i * 16, 16)]
            plsc.store_scatter(idx_v, [pos2], tok)
            plsc.store_scatter(ws_v, [pos2], w2_v[pl.ds(i * 16, 16)])
            return c

        lax.fori_loop(0, T // 16, sc_body, 0)

        pltpu.sync_copy(idx_v, idx_out)
        pltpu.sync_copy(ws_v, ws_out)


@functools.lru_cache(maxsize=None)
def _scatter_sc():
    return pl.kernel(
        _scatter_body,
        out_type=[
            jax.ShapeDtypeStruct((R_MAX,), jnp.int32),
            jax.ShapeDtypeStruct((R_MAX,), jnp.float32),
        ],
        mesh=_sc_mesh(),
        scratch_types=[
            pltpu.VMEM((T,), jnp.int32),
            pltpu.VMEM((T,), jnp.int32),
            pltpu.VMEM((T,), jnp.float32),
            pltpu.VMEM((T,), jnp.float32),
            pltpu.VMEM((R_MAX,), jnp.int32),
            pltpu.VMEM((R_MAX,), jnp.float32),
        ],
        compiler_params=_SC_PARAMS,
    )


def _mm_kernel(te_ref, p1_ref, p2_ref, w1c_ref, w2c_ref, x_ref, w1_ref,
               w3_ref, w2_ref, ys_ref):
    i = pl.program_id(0)

    @pl.when(te_ref[i] < E)
    def _():
        row = (lax.broadcasted_iota(jnp.int32, (TILE_R, T), 0)
               + i * TILE_R)
        sel1 = (p1_ref[...] == row).astype(jnp.float32)
        sel2 = (p2_ref[...] == row).astype(jnp.float32)
        sel = sel1 + sel2
        xb = jnp.dot(sel, x_ref[...], preferred_element_type=jnp.float32)
        wrow = (jnp.dot(sel1, w1c_ref[...],
                        preferred_element_type=jnp.float32)
                + jnp.dot(sel2, w2c_ref[...],
                          preferred_element_type=jnp.float32))
        a = jnp.dot(xb, w1_ref[0], preferred_element_type=jnp.float32)
        b = jnp.dot(xb, w3_ref[0], preferred_element_type=jnp.float32)
        h = a / (1.0 + jnp.exp(-a)) * b
        y = jnp.dot(h, w2_ref[0], preferred_element_type=jnp.float32)
        ys_ref[...] = y * wrow


def _w_index(i, te_ref):
    return (jnp.minimum(te_ref[i], E - 1), 0, 0)


_grouped_mm = pl.pallas_call(
    _mm_kernel,
    grid_spec=pltpu.PrefetchScalarGridSpec(
        num_scalar_prefetch=1,
        grid=(NT,),
        in_specs=[
            pl.BlockSpec((1, T), lambda i, te: (0, 0)),
            pl.BlockSpec((1, T), lambda i, te: (0, 0)),
            pl.BlockSpec((T, 1), lambda i, te: (0, 0)),
            pl.BlockSpec((T, 1), lambda i, te: (0, 0)),
            pl.BlockSpec((T, D_MODEL), lambda i, te: (0, 0)),
            pl.BlockSpec((1, D_MODEL, D_FF), _w_index),
            pl.BlockSpec((1, D_MODEL, D_FF), _w_index),
            pl.BlockSpec((1, D_FF, D_MODEL), _w_index),
        ],
        out_specs=pl.BlockSpec((TILE_R, D_MODEL), lambda i, te: (i, 0)),
    ),
    out_shape=jax.ShapeDtypeStruct((R_MAX, D_MODEL), jnp.float32),
)


_C_PER_W = T // NW


def _combine_body(p1_hbm, p2_hbm, ys_hbm, out_hbm, i1_v, i2_v, a_v, b_v,
                  sem1, sem2, sem3):
    wid = lax.axis_index("s") * 2 + lax.axis_index("c")
    base = wid * _C_PER_W
    pltpu.sync_copy(p1_hbm.at[pl.ds(base, _C_PER_W)], i1_v)
    pltpu.sync_copy(p2_hbm.at[pl.ds(base, _C_PER_W)], i2_v)
    cp1 = pltpu.async_copy(ys_hbm.at[i1_v], a_v, sem1)
    cp2 = pltpu.async_copy(ys_hbm.at[i2_v], b_v, sem2)
    cp1.wait()
    cp2.wait()

    def row(r, c2):
        for j in range(D_MODEL // 16):
            a_v[r, pl.ds(j * 16, 16)] = (a_v[r, pl.ds(j * 16, 16)]
                                         + b_v[r, pl.ds(j * 16, 16)])
        return c2

    lax.fori_loop(0, _C_PER_W, row, 0)
    pltpu.async_copy(a_v, out_hbm.at[pl.ds(base, _C_PER_W)], sem3).wait()


@functools.lru_cache(maxsize=None)
def _combine_sc():
    return pl.kernel(
        _combine_body,
        out_type=jax.ShapeDtypeStruct((T, D_MODEL), jnp.float32),
        mesh=_sc_mesh(),
        scratch_types=[
            pltpu.VMEM((_C_PER_W,), jnp.int32),
            pltpu.VMEM((_C_PER_W,), jnp.int32),
            pltpu.VMEM((_C_PER_W, D_MODEL), jnp.float32),
            pltpu.VMEM((_C_PER_W, D_MODEL), jnp.float32),
            pltpu.SemaphoreType.DMA,
            pltpu.SemaphoreType.DMA,
            pltpu.SemaphoreType.DMA,
        ],
        compiler_params=_SC_PARAMS,
    )


def kernel(hidden_states, Wg, steer_vector, W1, W3, W2):
    x = hidden_states
    p1, p2, w1, w2, te = _router(x, Wg, steer_vector.reshape(1, E))
    p1f = p1.reshape(T)
    p2f = p2.reshape(T)
    ys = _grouped_mm(te.reshape(NT), p1.reshape(1, T), p2.reshape(1, T),
                     w1, w2, x, W1, W3, W2)
    return _combine_sc()(p1f, p2f, ys)


# speedup vs baseline: 1.0875x; 1.0875x over previous
import functools

import jax
import jax.numpy as jnp
from jax import lax
from jax.experimental import pallas as pl
from jax.experimental.pallas import tpu as pltpu
from jax.experimental.pallas import tpu_sc as plsc

T, D_MODEL, D_FF, E, TOP_K = 2048, 768, 2048, 8, 2
TILE_R = 256
R_MAX = T * TOP_K + E * TILE_R
NT = R_MAX // TILE_R

NW = 32


@functools.lru_cache(maxsize=None)
def _sc_mesh():
    return plsc.VectorSubcoreMesh(core_axis_name="c", subcore_axis_name="s")


_SC_PARAMS = pltpu.CompilerParams(needs_layout_passes=False)


def _router_kernel(x_ref, wg_ref, steer_ref, p1_ref, p2_ref, w1_ref,
                   w2_ref, te_ref):
    x = x_ref[...]
    logits = jnp.dot(x, wg_ref[...], preferred_element_type=jnp.float32)
    logits = logits + steer_ref[...]
    eiota = lax.broadcasted_iota(jnp.int32, (T, E), 1)

    m1 = jnp.max(logits, axis=-1, keepdims=True)
    i1 = jnp.min(jnp.where(logits == m1, eiota, E), axis=-1, keepdims=True)
    masked = jnp.where(eiota == i1, jnp.float32(-1e30), logits)
    m2 = jnp.max(masked, axis=-1, keepdims=True)
    i2 = jnp.min(jnp.where(masked == m2, eiota, E), axis=-1, keepdims=True)

    w1 = 1.0 / (1.0 + jnp.exp(m2 - m1))
    w1_ref[...] = w1
    w2_ref[...] = 1.0 - w1

    cnt = ((eiota == i1) | (eiota == i2)).astype(jnp.float32)
    r_io = lax.broadcasted_iota(jnp.int32, (T, T), 0)
    c_io = lax.broadcasted_iota(jnp.int32, (T, T), 1)
    ltri = (r_io > c_io).astype(jnp.float32)
    exc = jnp.dot(ltri, cnt, preferred_element_type=jnp.float32)

    gs = jnp.sum(cnt, axis=0, keepdims=True)
    gs_pad = jnp.ceil(gs / TILE_R) * TILE_R
    ei = lax.broadcasted_iota(jnp.int32, (E, E), 0)
    ej = lax.broadcasted_iota(jnp.int32, (E, E), 1)
    upper = (ei < ej).astype(jnp.float32)
    off = jnp.dot(gs_pad, upper, preferred_element_type=jnp.float32)

    pos = off + exc
    p1 = jnp.sum(jnp.where(eiota == i1, pos, 0.0), axis=-1, keepdims=True)
    p2 = jnp.sum(jnp.where(eiota == i2, pos, 0.0), axis=-1, keepdims=True)
    p1_ref[...] = p1.astype(jnp.int32)
    p2_ref[...] = p2.astype(jnp.int32)

    pad_end = off + gs_pad
    t_start = (lax.broadcasted_iota(jnp.int32, (NT, 1), 0)
               * TILE_R).astype(jnp.float32)
    te_ref[...] = jnp.sum((t_start >= pad_end).astype(jnp.int32), axis=-1,
                          keepdims=True)


_router = pl.pallas_call(
    _router_kernel,
    out_shape=[
        jax.ShapeDtypeStruct((T, 1), jnp.int32),
        jax.ShapeDtypeStruct((T, 1), jnp.int32),
        jax.ShapeDtypeStruct((T, 1), jnp.float32),
        jax.ShapeDtypeStruct((T, 1), jnp.float32),
        jax.ShapeDtypeStruct((NT, 1), jnp.int32),
    ],
)


def _mm_kernel(te_ref, p1_ref, p2_ref, w1c_ref, w2c_ref, x_ref, w1_ref,
               w3_ref, w2_ref, ys_ref):
    i = pl.program_id(0)

    @pl.when(te_ref[i] < E)
    def _():
        row = (lax.broadcasted_iota(jnp.int32, (TILE_R, T), 0)
               + i * TILE_R)
        sel1 = (p1_ref[...] == row).astype(jnp.float32)
        sel2 = (p2_ref[...] == row).astype(jnp.float32)
        sel = sel1 + sel2
        xb = jnp.dot(sel, x_ref[...], preferred_element_type=jnp.float32)
        wrow = (jnp.dot(sel1, w1c_ref[...],
                        preferred_element_type=jnp.float32)
                + jnp.dot(sel2, w2c_ref[...],
                          preferred_element_type=jnp.float32))
        a = jnp.dot(xb, w1_ref[0], preferred_element_type=jnp.float32)
        b = jnp.dot(xb, w3_ref[0], preferred_element_type=jnp.float32)
        h = a / (1.0 + jnp.exp(-a)) * b
        y = jnp.dot(h, w2_ref[0], preferred_element_type=jnp.float32)
        ys_ref[...] = y * wrow


def _w_index(i, te_ref):
    return (jnp.minimum(te_ref[i], E - 1), 0, 0)


_grouped_mm = pl.pallas_call(
    _mm_kernel,
    grid_spec=pltpu.PrefetchScalarGridSpec(
        num_scalar_prefetch=1,
        grid=(NT,),
        in_specs=[
            pl.BlockSpec((1, T), lambda i, te: (0, 0)),
            pl.BlockSpec((1, T), lambda i, te: (0, 0)),
            pl.BlockSpec((T, 1), lambda i, te: (0, 0)),
            pl.BlockSpec((T, 1), lambda i, te: (0, 0)),
            pl.BlockSpec((T, D_MODEL), lambda i, te: (0, 0)),
            pl.BlockSpec((1, D_MODEL, D_FF), _w_index),
            pl.BlockSpec((1, D_MODEL, D_FF), _w_index),
            pl.BlockSpec((1, D_FF, D_MODEL), _w_index),
        ],
        out_specs=pl.BlockSpec((TILE_R, D_MODEL), lambda i, te: (i, 0)),
    ),
    out_shape=jax.ShapeDtypeStruct((R_MAX, D_MODEL), jnp.float32),
)


_C_PER_W = T // NW


def _combine_body(p1_hbm, p2_hbm, ys_hbm, out_hbm, i1_v, i2_v, a_v, b_v,
                  sem1, sem2, sem3):
    wid = lax.axis_index("s") * 2 + lax.axis_index("c")
    base = wid * _C_PER_W
    pltpu.sync_copy(p1_hbm.at[pl.ds(base, _C_PER_W)], i1_v)
    pltpu.sync_copy(p2_hbm.at[pl.ds(base, _C_PER_W)], i2_v)
    cp1 = pltpu.async_copy(ys_hbm.at[i1_v], a_v, sem1)
    cp2 = pltpu.async_copy(ys_hbm.at[i2_v], b_v, sem2)
    cp1.wait()
    cp2.wait()

    def row(r, c2):
        for j in range(D_MODEL // 16):
            a_v[r, pl.ds(j * 16, 16)] = (a_v[r, pl.ds(j * 16, 16)]
                                         + b_v[r, pl.ds(j * 16, 16)])
        return c2

    lax.fori_loop(0, _C_PER_W, row, 0)
    pltpu.async_copy(a_v, out_hbm.at[pl.ds(base, _C_PER_W)], sem3).wait()


@functools.lru_cache(maxsize=None)
def _combine_sc():
    return pl.kernel(
        _combine_body,
        out_type=jax.ShapeDtypeStruct((T, D_MODEL), jnp.float32),
        mesh=_sc_mesh(),
        scratch_types=[
            pltpu.VMEM((_C_PER_W,), jnp.int32),
            pltpu.VMEM((_C_PER_W,), jnp.int32),
            pltpu.VMEM((_C_PER_W, D_MODEL), jnp.float32),
            pltpu.VMEM((_C_PER_W, D_MODEL), jnp.float32),
            pltpu.SemaphoreType.DMA,
            pltpu.SemaphoreType.DMA,
            pltpu.SemaphoreType.DMA,
        ],
        compiler_params=_SC_PARAMS,
    )


def kernel(hidden_states, Wg, steer_vector, W1, W3, W2):
    x = hidden_states
    p1, p2, w1, w2, te = _router(x, Wg, steer_vector.reshape(1, E))
    p1f = p1.reshape(T)
    p2f = p2.reshape(T)
    ys = _grouped_mm(te.reshape(NT), p1.reshape(1, T), p2.reshape(1, T),
                     w1, w2, x, W1, W3, W2)
    return _combine_sc()(p1f, p2f, ys)
